# baseline (speedup 1.0000x reference)
import functools

import jax
import jax.numpy as jnp
from jax import lax
from jax.experimental import pallas as pl
from jax.experimental.pallas import tpu as pltpu
from jax.experimental.pallas import tpu_sc as plsc

N = 10000
E = 320000
ND = 128
ED = 16
H = 512
EPS = 1e-5

NC = 2
NS = 16
NW = NC * NS
LW = 128
IR = E // LW
IRP = 2560
EP = IRP * LW
RPT = IRP // NW
SRT = 160
CR = 5120
RT = CR + NS
EB = 16

BE = 2000
BN = 1000


def _gather_body(x_hbm, sidx_hbm, ridx_hbm, xs_hbm, xr_hbm,
                 sidx_v, ridx_v, bufs, gsem, wsem):
    c = lax.axis_index("c")
    s = lax.axis_index("s")
    wid = s * NC + c
    lo = wid * RPT
    pltpu.sync_copy(sidx_hbm.at[pl.ds(lo, RPT)], sidx_v)
    pltpu.sync_copy(ridx_hbm.at[pl.ds(lo, RPT)], ridx_v)

    T = 2 * RPT

    def fire(t, slot):
        j = t // 2

        @pl.when(lax.rem(t, 2) == 0)
        def _():
            pltpu.async_copy(x_hbm.at[sidx_v.at[j]], bufs.at[slot], gsem.at[slot])

        @pl.when(lax.rem(t, 2) == 1)
        def _():
            pltpu.async_copy(x_hbm.at[ridx_v.at[j]], bufs.at[slot], gsem.at[slot])

    pltpu.async_copy(x_hbm.at[sidx_v.at[0]], bufs.at[0], gsem.at[0])
    pltpu.async_copy(x_hbm.at[ridx_v.at[0]], bufs.at[1], gsem.at[1])

    def body(t, carry):
        slot = lax.rem(t, 4)
        j = t // 2
        row = lo + j
        nt = t + 2
        nslot = lax.rem(nt, 4)

        @pl.when(nt < T)
        def _():
            @pl.when(nt >= 4)
            def _():
                pltpu.make_async_copy(
                    x_hbm.at[pl.ds(0, LW)], bufs.at[nslot], wsem.at[nslot]
                ).wait()

            fire(nt, nslot)

        pltpu.make_async_copy(
            x_hbm.at[pl.ds(0, LW)], bufs.at[slot], gsem.at[slot]
        ).wait()

        @pl.when(lax.rem(t, 2) == 0)
        def _():
            pltpu.async_copy(bufs.at[slot], xs_hbm.at[pl.ds(row * LW, LW)],
                             wsem.at[slot])

        @pl.when(lax.rem(t, 2) == 1)
        def _():
            pltpu.async_copy(bufs.at[slot], xr_hbm.at[pl.ds(row * LW, LW)],
                             wsem.at[slot])

        return carry

    lax.fori_loop(0, T, body, 0)
    for sl in range(4):
        pltpu.make_async_copy(
            x_hbm.at[pl.ds(0, LW)], bufs.at[sl], wsem.at[sl]
        ).wait()


@functools.cache
def _make_gather():
    return pl.kernel(
        _gather_body,
        out_type=(
            jax.ShapeDtypeStruct((EP, ND), jnp.float32),
            jax.ShapeDtypeStruct((EP, ND), jnp.float32),
        ),
        mesh=plsc.VectorSubcoreMesh(
            core_axis_name="c", subcore_axis_name="s", num_cores=NC, num_subcores=NS
        ),
        scratch_types=[
            pltpu.VMEM((RPT, LW), jnp.int32),
            pltpu.VMEM((RPT, LW), jnp.int32),
            pltpu.VMEM((4, LW, ND), jnp.float32),
            pltpu.SemaphoreType.DMA((4,)),
            pltpu.SemaphoreType.DMA((4,)),
        ],
    )


def _gather(x, senders, receivers):
    return _make_gather()(x, senders, receivers)


def _edge_mlp_body(xs, xr, ea, w1a, w1b, w1c, b1, w2, b2, pay, stats):
    bf = jnp.bfloat16
    pre = (
        jnp.dot(xs[...].astype(bf), w1a[...], preferred_element_type=jnp.float32)
        + jnp.dot(xr[...].astype(bf), w1b[...], preferred_element_type=jnp.float32)
        + jnp.dot(ea[...].astype(bf), w1c[...], preferred_element_type=jnp.float32)
        + b1[...]
    )
    h = pre * jax.nn.sigmoid(pre)
    u = jnp.dot(h.astype(bf), w2[...], preferred_element_type=jnp.float32) + b2[...]
    ne = ea[...] + u
    pay[...] = jnp.concatenate(
        [u, jnp.ones((BE, 1), jnp.float32), jnp.zeros((BE, 15), jnp.float32),
         ne, jnp.zeros((BE, 80), jnp.float32)],
        axis=1,
    )
    s1 = jnp.sum(ne)
    s2 = jnp.sum(ne * ne)
    lane = lax.broadcasted_iota(jnp.int32, (1, 128), 1)
    vec = jnp.where(lane == 0, s1, 0.0) + jnp.where(lane == 1, s2, 0.0)

    @pl.when(pl.program_id(0) == 0)
    def _():
        stats[...] = jnp.zeros_like(stats)

    stats[...] += vec


def _edge_mlp(xs, xr, ea, w1a, w1b, w1c, b1, w2, b2):
    return pl.pallas_call(
        _edge_mlp_body,
        grid=(E // BE,),
        in_specs=[
            pl.BlockSpec((BE, ND), lambda i: (i, 0)),
            pl.BlockSpec((BE, ND), lambda i: (i, 0)),
            pl.BlockSpec((BE, ED), lambda i: (i, 0)),
            pl.BlockSpec((ND, H), lambda i: (0, 0)),
            pl.BlockSpec((ND, H), lambda i: (0, 0)),
            pl.BlockSpec((ED, H), lambda i: (0, 0)),
            pl.BlockSpec((1, H), lambda i: (0, 0)),
            pl.BlockSpec((H, ED), lambda i: (0, 0)),
            pl.BlockSpec((1, ED), lambda i: (0, 0)),
        ],
        out_specs=[
            pl.BlockSpec((BE, 128), lambda i: (i, 0)),
            pl.BlockSpec((1, 128), lambda i: (0, 0)),
        ],
        out_shape=[
            jax.ShapeDtypeStruct((E, 128), jnp.float32),
            jax.ShapeDtypeStruct((1, 128), jnp.float32),
        ],
    )(xs, xr, ea, w1a, w1b, w1c, b1, w2, b2)


def _scatter_body(pay_hbm, ridx_hbm, zeros_hbm, seg_hbm,
                  idx_all, cidx4, pay4, ebuf_v, seg_sh, psem, ssem):
    c = lax.axis_index("c")
    s = lax.axis_index("s")
    base = c * CR
    trash = CR + s

    @pl.when(s == 0)
    def _():
        pltpu.sync_copy(zeros_hbm, seg_sh)

    lo = s * SRT
    cnt = jnp.where(s < NS - 1, SRT, IR - (NS - 1) * SRT)
    pltpu.sync_copy(ridx_hbm.at[pl.ds(lo, SRT)], idx_all)
    plsc.subcore_barrier()

    pltpu.async_copy(pay_hbm.at[pl.ds(lo * LW, LW)], pay4.at[0], psem.at[0])
    pltpu.async_copy(pay_hbm.at[pl.ds((lo + 1) * LW, LW)], pay4.at[1], psem.at[1])

    def body(p, carry):
        slot = lax.rem(p, 4)
        nt = p + 2
        nslot = lax.rem(nt, 4)

        @pl.when(nt < cnt)
        def _():
            @pl.when(nt >= 4)
            def _():
                pltpu.make_async_copy(
                    pay_hbm.at[pl.ds(0, LW)], pay4.at[nslot], ssem.at[nslot]
                ).wait()

            pltpu.async_copy(pay_hbm.at[pl.ds((lo + nt) * LW, LW)],
                             pay4.at[nslot], psem.at[nslot])

        for k in range(LW // 16):
            v = idx_all[p, pl.ds(k * 16, 16)]
            loc = v - base
            ok = (loc >= 0) & (loc < CR)
            cidx4[slot, pl.ds(k * 16, 16)] = jnp.where(ok, loc, trash)

        pltpu.make_async_copy(
            pay_hbm.at[pl.ds(0, LW)], pay4.at[slot], psem.at[slot]
        ).wait()
        pltpu.async_copy(pay4.at[slot], seg_sh.at[cidx4.at[slot]],
                         ssem.at[slot], add=True)
        return carry

    lax.fori_loop(0, cnt, body, 0)
    for sl in range(4):
        @pl.when(cnt - 4 + sl >= 0)
        def _():
            pltpu.make_async_copy(
                pay_hbm.at[pl.ds(0, LW)], pay4.at[(0 + sl) % 4], ssem.at[sl]
            ).wait()
    plsc.subcore_barrier()

    def ebody(k, carry):
        b = s * (CR // NS) + k * EB
        pltpu.sync_copy(seg_sh.at[pl.ds(b, EB)], ebuf_v)
        pltpu.sync_copy(ebuf_v, seg_hbm.at[c, pl.ds(b, EB)])
        return carry

    lax.fori_loop(0, CR // NS // EB, ebody, 0)


@functools.cache
def _make_scatter():
    return pl.kernel(
        _scatter_body,
        out_type=jax.ShapeDtypeStruct((NC, CR, 128), jnp.float32),
        mesh=plsc.VectorSubcoreMesh(
            core_axis_name="c", subcore_axis_name="s", num_cores=NC, num_subcores=NS
        ),
        scratch_types=[
            pltpu.VMEM((SRT, LW), jnp.int32),
            pltpu.VMEM((4, LW), jnp.int32),
            pltpu.VMEM((4, LW, 128), jnp.float32),
            pltpu.VMEM((EB, 128), jnp.float32),
            pltpu.VMEM_SHARED((RT, 128), jnp.float32),
            pltpu.SemaphoreType.DMA((4,)),
            pltpu.SemaphoreType.DMA((4,)),
        ],
    )


def _scatter(pay, receivers, zeros_init):
    return _make_scatter()(pay, receivers, zeros_init)


def _edge_norm_body(pay, stats, w, b, out):
    st = stats[...]
    lane = lax.broadcasted_iota(jnp.int32, (1, 128), 1)
    tot = jnp.float32(E * ED)
    s1 = jnp.sum(jnp.where(lane == 0, st, 0.0))
    s2 = jnp.sum(jnp.where(lane == 1, st, 0.0))
    mean = s1 / tot
    var = jnp.maximum(s2 / tot - mean * mean, 0.0)
    inv = 1.0 / (jnp.sqrt(var) + EPS)
    ne = pay[:, 32:32 + ED]
    out[...] = (ne - mean) * inv * w[...] + b[...]


def _edge_norm(pay, stats, w, b):
    return pl.pallas_call(
        _edge_norm_body,
        grid=(E // BE,),
        in_specs=[
            pl.BlockSpec((BE, 128), lambda i: (i, 0)),
            pl.BlockSpec((1, 128), lambda i: (0, 0)),
            pl.BlockSpec((1, ED), lambda i: (0, 0)),
            pl.BlockSpec((1, ED), lambda i: (0, 0)),
        ],
        out_specs=pl.BlockSpec((BE, ED), lambda i: (i, 0)),
        out_shape=jax.ShapeDtypeStruct((E, ED), jnp.float32),
    )(pay, stats, w, b)


def _node_body(x, segp, w1a, w1b, b1, w2, b2, lw, lb, out):
    sp = segp[...]
    agg = sp[:, :ED] / jnp.maximum(sp[:, ED:ED + 1], 1.0)
    pre = (
        jnp.dot(x[...], w1a[...], preferred_element_type=jnp.float32)
        + jnp.dot(agg, w1b[...], preferred_element_type=jnp.float32)
        + b1[...]
    )
    g = pre * jax.nn.sigmoid(pre)
    u = jnp.dot(g, w2[...], preferred_element_type=jnp.float32) + b2[...]
    nx = x[...] + u
    mu = jnp.mean(nx, axis=-1, keepdims=True)
    d = nx - mu
    var = jnp.mean(d * d, axis=-1, keepdims=True)
    out[...] = d * lax.rsqrt(var + EPS) * lw[...] + lb[...]


def _node_mlp(x, segp, w1a, w1b, b1, w2, b2, lw, lb):
    return pl.pallas_call(
        _node_body,
        grid=(N // BN,),
        in_specs=[
            pl.BlockSpec((BN, ND), lambda i: (i, 0)),
            pl.BlockSpec((BN, 128), lambda i: (i, 0)),
            pl.BlockSpec((ND, H), lambda i: (0, 0)),
            pl.BlockSpec((ED, H), lambda i: (0, 0)),
            pl.BlockSpec((1, H), lambda i: (0, 0)),
            pl.BlockSpec((H, ND), lambda i: (0, 0)),
            pl.BlockSpec((1, ND), lambda i: (0, 0)),
            pl.BlockSpec((1, ND), lambda i: (0, 0)),
            pl.BlockSpec((1, ND), lambda i: (0, 0)),
        ],
        out_specs=pl.BlockSpec((BN, ND), lambda i: (i, 0)),
        out_shape=jax.ShapeDtypeStruct((N, ND), jnp.float32),
    )(x, segp, w1a, w1b, b1, w2, b2, lw, lb)


def kernel(x, edge_index, edge_attr, ew1, eb1, ew2, eb2, nw1, nb1, nw2, nb2,
           eln_w, eln_b, nln_w, nln_b):
    senders_p = jnp.pad(edge_index[0], (0, EP - E)).reshape(IRP, LW)
    receivers_p = jnp.pad(edge_index[1], (0, EP - E)).reshape(IRP, LW)

    xs, xr = _gather(x, senders_p, receivers_p)

    bf = jnp.bfloat16
    pay, stats = _edge_mlp(
        xs, xr, edge_attr,
        ew1[:ND].astype(bf), ew1[ND:2 * ND].astype(bf), ew1[2 * ND:].astype(bf),
        eb1.reshape(1, H), ew2.astype(bf), eb2.reshape(1, ED),
    )

    zeros_init = jnp.zeros((RT, 128), dtype=jnp.float32)
    seg = _scatter(pay, receivers_p, zeros_init)
    segp = jnp.concatenate([seg[0], seg[1]], axis=0)[:N]

    new_edge_attr = _edge_norm(
        pay, stats, eln_w.reshape(1, ED), eln_b.reshape(1, ED)
    )

    new_x = _node_mlp(
        x, segp,
        nw1[:ND], nw1[ND:], nb1.reshape(1, H), nw2, nb2.reshape(1, ND),
        nln_w.reshape(1, ND), nln_b.reshape(1, ND),
    )
    return new_x, new_edge_attr
